# baseline (speedup 1.0000x reference)
import jax
import jax.numpy as jnp
from jax import lax
from jax.experimental import pallas as pl
from jax.experimental.pallas import tpu as pltpu
from jax.experimental.pallas import tpu_sc as plsc

_C, _G, _S, _L = 16, 4096, 8, 2
_B = 32
_STEPS = 3
_GAMMA = 0.01
_IG = 100.0
_IG2 = 144.26950408889634
_LN2 = 0.6931471805599453
_C1 = _GAMMA * _LN2
_C2 = 2.0 * _GAMMA

_NC, _NS = 2, 16
_NW = _NC * _NS
_SLOTS = _C * _G
_SPW = _SLOTS // _NW
_CS = 4
_RPC = _CS * _S * _L
_CPW = _SPW // _CS
_OSL = 256
_CPO = _OSL // _CS
_OBW = _SPW // _OSL
_NROWS = _SLOTS * _S * _L // _RPC


def _p1(f, xs, *cs):
    return [f(x, *cs) for x in xs]


def _p2(f, xs, ys):
    return [f(x, y) for x, y in zip(xs, ys)]


def _ptree(f, pairs_list):
    while len(pairs_list) > 1:
        nxt = [_p2(f, pairs_list[i], pairs_list[i + 1])
               for i in range(0, len(pairs_list) - 1, 2)]
        if len(pairs_list) % 2:
            nxt.append(pairs_list[-1])
        pairs_list = nxt
    return pairs_list[0]


def _compute_chunk(rows_v, outa_v, col):
    for k2 in range(0, _CS, 2):
        g4 = [(k2, 0), (k2, 16), (k2 + 1, 0), (k2 + 1, 16)]
        acc = None
        for s in range(_S):
            r0 = [rows_v[k * 16 + 2 * s, pl.ds(lo, 16)] for k, lo in g4]
            r1 = [rows_v[k * 16 + 2 * s + 1, pl.ds(lo, 16)] for k, lo in g4]
            b = _p2(lambda a, bb: a * bb, r0, r1)
            e = _p1(lambda bb: jnp.exp((bb - 1.0) * _IG), b)
            acc = e if acc is None else _p2(lambda a, bb: a + bb, acc, e)
        for i, (k, lo) in enumerate(g4):
            outa_v[pl.ds((col + k) * _B + lo, 16)] = acc[i]


def _stage_a_body(idx_hbm, xt_hbm, pa_hbm,
                  idx_v, rows_a, rows_b, outa_v, sem_a, sem_b):
    cid = lax.axis_index("c")
    sid = lax.axis_index("s")
    w = sid * _NC + cid
    cc = w // 2
    gb = (w % 2) * (_G // 2)

    pltpu.sync_copy(idx_hbm.at[pl.ds(w * _CPW, _CPW), :], idx_v)

    def issue(ch, rows, sem):
        pltpu.async_copy(xt_hbm.at[idx_v.at[ch]], rows, sem)

    def wait(rows, sem):
        pltpu.make_async_copy(xt_hbm.at[idx_v.at[0]], rows, sem).wait()

    def ob_body(ob, carry):
        c0 = ob * _CPO
        issue(c0, rows_a, sem_a)

        def pair_body(p, c_):
            j0 = c0 + p * 2
            issue(j0 + 1, rows_b, sem_b)
            wait(rows_a, sem_a)
            _compute_chunk(rows_a, outa_v, (p * 2) * _CS)

            @pl.when(p < _CPO // 2 - 1)
            def _():
                issue(j0 + 2, rows_a, sem_a)

            wait(rows_b, sem_b)
            _compute_chunk(rows_b, outa_v, (p * 2 + 1) * _CS)
            return c_

        lax.fori_loop(0, _CPO // 2, pair_body, 0)
        off = ((cc * _G + gb) + ob * _OSL) * _B
        pltpu.sync_copy(outa_v, pa_hbm.at[pl.ds(off, _OSL * _B)])
        return carry

    lax.fori_loop(0, _OBW, ob_body, 0)


_stage_a = pl.kernel(
    _stage_a_body,
    out_type=jax.ShapeDtypeStruct((_C * _G * _B,), jnp.float32),
    mesh=plsc.VectorSubcoreMesh(core_axis_name="c", subcore_axis_name="s"),
    compiler_params=pltpu.CompilerParams(use_tc_tiling_on_sc=False),
    scratch_types=(
        pltpu.VMEM((_CPW, _RPC), jnp.int32),
        pltpu.VMEM((_RPC, _B), jnp.float32),
        pltpu.VMEM((_RPC, _B), jnp.float32),
        pltpu.VMEM((_OSL * _B,), jnp.float32),
        pltpu.SemaphoreType.DMA,
        pltpu.SemaphoreType.DMA,
    ),
)


def _stage_b_body(pa_ref, rt_ref, rnt_ref):
    lse_s = 1.0 + _GAMMA * jnp.log(pa_ref[...])
    m1 = jnp.max(lse_s)
    s1 = jnp.where(m1 > 1.0, 1.0 / m1, 1.0)
    cv = lse_s * s1
    mxc = jnp.maximum(jnp.max(cv, axis=0), -1e30)
    acc = jnp.sum(jnp.exp((cv - mxc[None, :, :]) * _IG), axis=0)
    lse_c = mxc + _GAMMA * jnp.log(acc)
    m2 = jnp.max(lse_c)
    rr = lse_c * jnp.where(m2 > 1.0, 1.0 / m2, 1.0)
    rc = rt_ref[...]
    mx2 = jnp.maximum(rc, rr)
    z = mx2 + _GAMMA * jnp.log(jnp.exp((rc - mx2) * _IG)
                               + jnp.exp((rr - mx2) * _IG))
    m3 = jnp.max(z)
    rnt_ref[...] = z * jnp.where(m3 > 1.0, 1.0 / m3, 1.0)


_GB = _G * _B
_ROWS128 = _GB // 128

_stage_b = pl.pallas_call(
    _stage_b_body,
    out_shape=jax.ShapeDtypeStruct((_ROWS128, 128), jnp.float32),
)


def _tr_body(rt_ref, r_ref):
    r_ref[...] = rt_ref[...].T


_tr = pl.pallas_call(
    _tr_body,
    out_shape=jax.ShapeDtypeStruct((_B, _G), jnp.float32),
)


def kernel(x, I):
    idx = I.reshape(_NROWS, _RPC).astype(jnp.int32)
    rt = x.T
    for _ in range(_STEPS):
        pa = _stage_a(idx, rt)
        rtf = _stage_b(pa.reshape(_C, _ROWS128, 128),
                       rt.reshape(_ROWS128, 128))
        rt = rtf.reshape(_G, _B)
    return _tr(rt)


# speedup vs baseline: 1.2702x; 1.2702x over previous
import jax
import jax.numpy as jnp
from jax import lax
from jax.experimental import pallas as pl
from jax.experimental.pallas import tpu as pltpu
from jax.experimental.pallas import tpu_sc as plsc

_C, _G, _S, _L = 16, 4096, 8, 2
_B = 32
_STEPS = 3
_GAMMA = 0.01
_IG = 100.0
_IG2 = 144.26950408889634
_LN2 = 0.6931471805599453
_C1 = _GAMMA * _LN2
_C2 = 2.0 * _GAMMA

_NC, _NS = 2, 16
_NW = _NC * _NS
_SLOTS = _C * _G
_SPW = _SLOTS // _NW
_CS = 8
_RPC = _CS * _S * _L
_CPW = _SPW // _CS
_OSL = 512
_CPO = _OSL // _CS
_OBW = _SPW // _OSL
_NROWS = _SLOTS * _S * _L // _RPC


def _p1(f, xs, *cs):
    return [f(x, *cs) for x in xs]


def _p2(f, xs, ys):
    return [f(x, y) for x, y in zip(xs, ys)]


def _ptree(f, pairs_list):
    while len(pairs_list) > 1:
        nxt = [_p2(f, pairs_list[i], pairs_list[i + 1])
               for i in range(0, len(pairs_list) - 1, 2)]
        if len(pairs_list) % 2:
            nxt.append(pairs_list[-1])
        pairs_list = nxt
    return pairs_list[0]


def _compute_chunk(rows_v, outa_v, col):
    for k2 in range(0, _CS, 2):
        g4 = [(k2, 0), (k2, 16), (k2 + 1, 0), (k2 + 1, 16)]
        acc = None
        for s in range(_S):
            r0 = [rows_v[k * 16 + 2 * s, pl.ds(lo, 16)] for k, lo in g4]
            r1 = [rows_v[k * 16 + 2 * s + 1, pl.ds(lo, 16)] for k, lo in g4]
            b = _p2(lambda a, bb: a * bb, r0, r1)
            e = _p1(lambda bb: jnp.exp((bb - 1.0) * _IG), b)
            acc = e if acc is None else _p2(lambda a, bb: a + bb, acc, e)
        for i, (k, lo) in enumerate(g4):
            outa_v[pl.ds((col + k) * _B + lo, 16)] = acc[i]


def _stage_a_body(idx_hbm, xt_hbm, pa_hbm,
                  idx_v, rows_a, rows_b, outa_v, sem_a, sem_b):
    cid = lax.axis_index("c")
    sid = lax.axis_index("s")
    w = sid * _NC + cid
    cc = w // 2
    gb = (w % 2) * (_G // 2)

    pltpu.sync_copy(idx_hbm.at[pl.ds(w * _CPW, _CPW), :], idx_v)

    def issue(ch, rows, sem):
        pltpu.async_copy(xt_hbm.at[idx_v.at[ch]], rows, sem)

    def wait(rows, sem):
        pltpu.make_async_copy(xt_hbm.at[idx_v.at[0]], rows, sem).wait()

    def ob_body(ob, carry):
        c0 = ob * _CPO
        issue(c0, rows_a, sem_a)

        def pair_body(p, c_):
            j0 = c0 + p * 2
            issue(j0 + 1, rows_b, sem_b)
            wait(rows_a, sem_a)
            _compute_chunk(rows_a, outa_v, (p * 2) * _CS)

            @pl.when(p < _CPO // 2 - 1)
            def _():
                issue(j0 + 2, rows_a, sem_a)

            wait(rows_b, sem_b)
            _compute_chunk(rows_b, outa_v, (p * 2 + 1) * _CS)
            return c_

        lax.fori_loop(0, _CPO // 2, pair_body, 0)
        off = ((cc * _G + gb) + ob * _OSL) * _B
        pltpu.sync_copy(outa_v, pa_hbm.at[pl.ds(off, _OSL * _B)])
        return carry

    lax.fori_loop(0, _OBW, ob_body, 0)


_stage_a = pl.kernel(
    _stage_a_body,
    out_type=jax.ShapeDtypeStruct((_C * _G * _B,), jnp.float32),
    mesh=plsc.VectorSubcoreMesh(core_axis_name="c", subcore_axis_name="s"),
    compiler_params=pltpu.CompilerParams(use_tc_tiling_on_sc=False),
    scratch_types=(
        pltpu.VMEM((_CPW, _RPC), jnp.int32),
        pltpu.VMEM((_RPC, _B), jnp.float32),
        pltpu.VMEM((_RPC, _B), jnp.float32),
        pltpu.VMEM((_OSL * _B,), jnp.float32),
        pltpu.SemaphoreType.DMA,
        pltpu.SemaphoreType.DMA,
    ),
)


def _stage_b_body(pa_ref, rt_ref, rnt_ref):
    lse_s = 1.0 + _GAMMA * jnp.log(pa_ref[...])
    m1 = jnp.max(lse_s)
    s1 = jnp.where(m1 > 1.0, 1.0 / m1, 1.0)
    cv = lse_s * s1
    mxc = jnp.maximum(jnp.max(cv, axis=0), -1e30)
    acc = jnp.sum(jnp.exp((cv - mxc[None, :, :]) * _IG), axis=0)
    lse_c = mxc + _GAMMA * jnp.log(acc)
    m2 = jnp.max(lse_c)
    rr = lse_c * jnp.where(m2 > 1.0, 1.0 / m2, 1.0)
    rc = rt_ref[...]
    mx2 = jnp.maximum(rc, rr)
    z = mx2 + _GAMMA * jnp.log(jnp.exp((rc - mx2) * _IG)
                               + jnp.exp((rr - mx2) * _IG))
    m3 = jnp.max(z)
    rnt_ref[...] = z * jnp.where(m3 > 1.0, 1.0 / m3, 1.0)


_GB = _G * _B
_ROWS128 = _GB // 128

_stage_b = pl.pallas_call(
    _stage_b_body,
    out_shape=jax.ShapeDtypeStruct((_ROWS128, 128), jnp.float32),
)


def _tr_body(rt_ref, r_ref):
    r_ref[...] = rt_ref[...].T


_tr = pl.pallas_call(
    _tr_body,
    out_shape=jax.ShapeDtypeStruct((_B, _G), jnp.float32),
)


def kernel(x, I):
    idx = I.reshape(_NROWS, _RPC).astype(jnp.int32)
    rt = x.T
    for _ in range(_STEPS):
        pa = _stage_a(idx, rt)
        rtf = _stage_b(pa.reshape(_C, _ROWS128, 128),
                       rt.reshape(_ROWS128, 128))
        rt = rtf.reshape(_G, _B)
    return _tr(rt)


# speedup vs baseline: 1.2819x; 1.0093x over previous
import jax
import jax.numpy as jnp
from jax import lax
from jax.experimental import pallas as pl
from jax.experimental.pallas import tpu as pltpu
from jax.experimental.pallas import tpu_sc as plsc

_C, _G, _S, _L = 16, 4096, 8, 2
_B = 32
_STEPS = 3
_GAMMA = 0.01
_IG = 100.0
_IG2 = 144.26950408889634
_LN2 = 0.6931471805599453
_C1 = _GAMMA * _LN2
_C2 = 2.0 * _GAMMA

_NC, _NS = 2, 16
_NW = _NC * _NS
_SLOTS = _C * _G
_SPW = _SLOTS // _NW
_CS = 8
_RPC = _CS * _S * _L
_CPW = _SPW // _CS
_OSL = 2048
_CPO = _OSL // _CS
_OBW = _SPW // _OSL
_NROWS = _SLOTS * _S * _L // _RPC


def _p1(f, xs, *cs):
    return [f(x, *cs) for x in xs]


def _p2(f, xs, ys):
    return [f(x, y) for x, y in zip(xs, ys)]


def _ptree(f, pairs_list):
    while len(pairs_list) > 1:
        nxt = [_p2(f, pairs_list[i], pairs_list[i + 1])
               for i in range(0, len(pairs_list) - 1, 2)]
        if len(pairs_list) % 2:
            nxt.append(pairs_list[-1])
        pairs_list = nxt
    return pairs_list[0]


def _compute_chunk(rows_v, outa_v, col):
    for k2 in range(0, _CS, 2):
        g4 = [(k2, 0), (k2, 16), (k2 + 1, 0), (k2 + 1, 16)]
        acc = None
        for s in range(_S):
            r0 = [rows_v[k * 16 + 2 * s, pl.ds(lo, 16)] for k, lo in g4]
            r1 = [rows_v[k * 16 + 2 * s + 1, pl.ds(lo, 16)] for k, lo in g4]
            b = _p2(lambda a, bb: a * bb, r0, r1)
            e = _p1(lambda bb: jnp.exp((bb - 1.0) * _IG), b)
            acc = e if acc is None else _p2(lambda a, bb: a + bb, acc, e)
        for i, (k, lo) in enumerate(g4):
            outa_v[pl.ds((col + k) * _B + lo, 16)] = acc[i]


def _stage_a_body(idx_hbm, xt_hbm, pa_hbm,
                  idx_v, rows_a, rows_b, outa_v, sem_a, sem_b):
    cid = lax.axis_index("c")
    sid = lax.axis_index("s")
    w = sid * _NC + cid
    cc = w // 2
    gb = (w % 2) * (_G // 2)

    pltpu.sync_copy(idx_hbm.at[pl.ds(w * _CPW, _CPW), :], idx_v)

    def issue(ch, rows, sem):
        pltpu.async_copy(xt_hbm.at[idx_v.at[ch]], rows, sem)

    def wait(rows, sem):
        pltpu.make_async_copy(xt_hbm.at[idx_v.at[0]], rows, sem).wait()

    def ob_body(ob, carry):
        c0 = ob * _CPO
        issue(c0, rows_a, sem_a)

        def pair_body(p, c_):
            j0 = c0 + p * 2
            issue(j0 + 1, rows_b, sem_b)
            wait(rows_a, sem_a)
            _compute_chunk(rows_a, outa_v, (p * 2) * _CS)

            @pl.when(p < _CPO // 2 - 1)
            def _():
                issue(j0 + 2, rows_a, sem_a)

            wait(rows_b, sem_b)
            _compute_chunk(rows_b, outa_v, (p * 2 + 1) * _CS)
            return c_

        lax.fori_loop(0, _CPO // 2, pair_body, 0)
        off = ((cc * _G + gb) + ob * _OSL) * _B
        pltpu.sync_copy(outa_v, pa_hbm.at[pl.ds(off, _OSL * _B)])
        return carry

    lax.fori_loop(0, _OBW, ob_body, 0)


_stage_a = pl.kernel(
    _stage_a_body,
    out_type=jax.ShapeDtypeStruct((_C * _G * _B,), jnp.float32),
    mesh=plsc.VectorSubcoreMesh(core_axis_name="c", subcore_axis_name="s"),
    compiler_params=pltpu.CompilerParams(use_tc_tiling_on_sc=False),
    scratch_types=(
        pltpu.VMEM((_CPW, _RPC), jnp.int32),
        pltpu.VMEM((_RPC, _B), jnp.float32),
        pltpu.VMEM((_RPC, _B), jnp.float32),
        pltpu.VMEM((_OSL * _B,), jnp.float32),
        pltpu.SemaphoreType.DMA,
        pltpu.SemaphoreType.DMA,
    ),
)


def _stage_b_body(pa_ref, rt_ref, rnt_ref):
    lse_s = 1.0 + _GAMMA * jnp.log(pa_ref[...])
    m1 = jnp.max(lse_s)
    s1 = jnp.where(m1 > 1.0, 1.0 / m1, 1.0)
    cv = lse_s * s1
    mxc = jnp.maximum(jnp.max(cv, axis=0), -1e30)
    acc = jnp.sum(jnp.exp((cv - mxc[None, :, :]) * _IG), axis=0)
    lse_c = mxc + _GAMMA * jnp.log(acc)
    m2 = jnp.max(lse_c)
    rr = lse_c * jnp.where(m2 > 1.0, 1.0 / m2, 1.0)
    rc = rt_ref[...]
    mx2 = jnp.maximum(rc, rr)
    z = mx2 + _GAMMA * jnp.log(jnp.exp((rc - mx2) * _IG)
                               + jnp.exp((rr - mx2) * _IG))
    m3 = jnp.max(z)
    rnt_ref[...] = z * jnp.where(m3 > 1.0, 1.0 / m3, 1.0)


_GB = _G * _B
_ROWS128 = _GB // 128

_stage_b = pl.pallas_call(
    _stage_b_body,
    out_shape=jax.ShapeDtypeStruct((_ROWS128, 128), jnp.float32),
)


def _tr_body(rt_ref, r_ref):
    r_ref[...] = rt_ref[...].T


_tr = pl.pallas_call(
    _tr_body,
    out_shape=jax.ShapeDtypeStruct((_B, _G), jnp.float32),
)


def kernel(x, I):
    idx = I.reshape(_NROWS, _RPC).astype(jnp.int32)
    rt = x.T
    for _ in range(_STEPS):
        pa = _stage_a(idx, rt)
        rtf = _stage_b(pa.reshape(_C, _ROWS128, 128),
                       rt.reshape(_ROWS128, 128))
        rt = rtf.reshape(_G, _B)
    return _tr(rt)


# speedup vs baseline: 1.3160x; 1.0266x over previous
import jax
import jax.numpy as jnp
from jax import lax
from jax.experimental import pallas as pl
from jax.experimental.pallas import tpu as pltpu
from jax.experimental.pallas import tpu_sc as plsc

_C, _G, _S, _L = 16, 4096, 8, 2
_B = 32
_STEPS = 3
_GAMMA = 0.01
_IG = 100.0
_IG2 = 144.26950408889634
_LN2 = 0.6931471805599453
_C1 = _GAMMA * _LN2
_C2 = 2.0 * _GAMMA

_NC, _NS = 2, 16
_NW = _NC * _NS
_SLOTS = _C * _G
_SPW = _SLOTS // _NW
_CS = 8
_RPC = _CS * _S * _L
_CPW = _SPW // _CS
_OSL = 2048
_CPO = _OSL // _CS
_OBW = _SPW // _OSL
_NROWS = _SLOTS * _S * _L // _RPC


def _p1(f, xs, *cs):
    return [f(x, *cs) for x in xs]


def _p2(f, xs, ys):
    return [f(x, y) for x, y in zip(xs, ys)]


def _ptree(f, pairs_list):
    while len(pairs_list) > 1:
        nxt = [_p2(f, pairs_list[i], pairs_list[i + 1])
               for i in range(0, len(pairs_list) - 1, 2)]
        if len(pairs_list) % 2:
            nxt.append(pairs_list[-1])
        pairs_list = nxt
    return pairs_list[0]


def _compute_chunk(rows_v, outa_v, col):
    gall = [(k, lo) for k in range(_CS) for lo in (0, 16)]
    acc = None
    for s in range(_S):
        r0 = [rows_v[k * 16 + 2 * s, pl.ds(lo, 16)] for k, lo in gall]
        r1 = [rows_v[k * 16 + 2 * s + 1, pl.ds(lo, 16)] for k, lo in gall]
        b = _p2(lambda a, bb: a * bb, r0, r1)
        e = _p1(lambda bb: jnp.exp((bb - 1.0) * _IG), b)
        acc = e if acc is None else _p2(lambda a, bb: a + bb, acc, e)
    for i, (k, lo) in enumerate(gall):
        outa_v[pl.ds((col + k) * _B + lo, 16)] = acc[i]


def _stage_a_body(idx_hbm, xt_hbm, pa_hbm,
                  idx_v, rows_a, rows_b, outa_v, sem_a, sem_b):
    cid = lax.axis_index("c")
    sid = lax.axis_index("s")
    w = sid * _NC + cid
    cc = w // 2
    gb = (w % 2) * (_G // 2)

    pltpu.sync_copy(idx_hbm.at[pl.ds(w * _CPW, _CPW), :], idx_v)

    def issue(ch, rows, sem):
        pltpu.async_copy(xt_hbm.at[idx_v.at[ch]], rows, sem)

    def wait(rows, sem):
        pltpu.make_async_copy(xt_hbm.at[idx_v.at[0]], rows, sem).wait()

    def ob_body(ob, carry):
        c0 = ob * _CPO
        issue(c0, rows_a, sem_a)

        def pair_body(p, c_):
            j0 = c0 + p * 2
            issue(j0 + 1, rows_b, sem_b)
            wait(rows_a, sem_a)
            _compute_chunk(rows_a, outa_v, (p * 2) * _CS)

            @pl.when(p < _CPO // 2 - 1)
            def _():
                issue(j0 + 2, rows_a, sem_a)

            wait(rows_b, sem_b)
            _compute_chunk(rows_b, outa_v, (p * 2 + 1) * _CS)
            return c_

        lax.fori_loop(0, _CPO // 2, pair_body, 0)
        off = ((cc * _G + gb) + ob * _OSL) * _B
        pltpu.sync_copy(outa_v, pa_hbm.at[pl.ds(off, _OSL * _B)])
        return carry

    lax.fori_loop(0, _OBW, ob_body, 0)


_stage_a = pl.kernel(
    _stage_a_body,
    out_type=jax.ShapeDtypeStruct((_C * _G * _B,), jnp.float32),
    mesh=plsc.VectorSubcoreMesh(core_axis_name="c", subcore_axis_name="s"),
    compiler_params=pltpu.CompilerParams(use_tc_tiling_on_sc=False),
    scratch_types=(
        pltpu.VMEM((_CPW, _RPC), jnp.int32),
        pltpu.VMEM((_RPC, _B), jnp.float32),
        pltpu.VMEM((_RPC, _B), jnp.float32),
        pltpu.VMEM((_OSL * _B,), jnp.float32),
        pltpu.SemaphoreType.DMA,
        pltpu.SemaphoreType.DMA,
    ),
)


def _stage_b_body(pa_ref, rt_ref, rnt_ref):
    lse_s = 1.0 + _GAMMA * jnp.log(pa_ref[...])
    m1 = jnp.max(lse_s)
    s1 = jnp.where(m1 > 1.0, 1.0 / m1, 1.0)
    cv = lse_s * s1
    mxc = jnp.maximum(jnp.max(cv, axis=0), -1e30)
    acc = jnp.sum(jnp.exp((cv - mxc[None, :, :]) * _IG), axis=0)
    lse_c = mxc + _GAMMA * jnp.log(acc)
    m2 = jnp.max(lse_c)
    rr = lse_c * jnp.where(m2 > 1.0, 1.0 / m2, 1.0)
    rc = rt_ref[...]
    mx2 = jnp.maximum(rc, rr)
    z = mx2 + _GAMMA * jnp.log(jnp.exp((rc - mx2) * _IG)
                               + jnp.exp((rr - mx2) * _IG))
    m3 = jnp.max(z)
    rnt_ref[...] = z * jnp.where(m3 > 1.0, 1.0 / m3, 1.0)


_GB = _G * _B
_ROWS128 = _GB // 128

_stage_b = pl.pallas_call(
    _stage_b_body,
    out_shape=jax.ShapeDtypeStruct((_ROWS128, 128), jnp.float32),
)


def _tr_body(rt_ref, r_ref):
    r_ref[...] = rt_ref[...].T


_tr = pl.pallas_call(
    _tr_body,
    out_shape=jax.ShapeDtypeStruct((_B, _G), jnp.float32),
)


def kernel(x, I):
    idx = I.reshape(_NROWS, _RPC).astype(jnp.int32)
    rt = x.T
    for _ in range(_STEPS):
        pa = _stage_a(idx, rt)
        rtf = _stage_b(pa.reshape(_C, _ROWS128, 128),
                       rt.reshape(_ROWS128, 128))
        rt = rtf.reshape(_G, _B)
    return _tr(rt)
